# Initial kernel scaffold; baseline (speedup 1.0000x reference)
import jax
import jax.numpy as jnp
from jax.experimental import pallas as pl


def kernel(nodes, edge_indices, edge_attr, global_attr, num_nodes, num_edges, batch_indices, target_batch_wide, W_msg, b_msg, W_upd, b_upd, W_glob, b_glob, W_act, b_act):
    raise NotImplementedError("write your pallas kernel here")


# speedup vs baseline: 1.4517x; 1.4517x over previous
import functools

import jax
import jax.numpy as jnp
from jax import lax
from jax.experimental import pallas as pl
from jax.experimental.pallas import tpu as pltpu

N = 10000
E = 320000
DF = 128
DM = 128
NEG = -3.0e38


def _prep_body(nodes_ref, glob_ref, w13_ref, w4_ref, bmsg_ref, s_ref, tg_ref):
    sm = jnp.dot(nodes_ref[...], w13_ref[...], preferred_element_type=jnp.float32)
    g = jnp.dot(glob_ref[...], w4_ref[...], preferred_element_type=jnp.float32)
    g = g + bmsg_ref[...][None, :]
    s_ref[...] = sm[:, :DM]
    tg_ref[...] = sm[:, DM:] + g


def _edgeproj_body(ea_ref, w2_ref, a_ref):
    a_ref[...] = jnp.dot(ea_ref[...], w2_ref[...], preferred_element_type=jnp.float32)


def _post_body(nodes_ref, m_ref, tg_ref, glob_ref, wu1_ref, wu2_ref, wu3_ref,
               bu_ref, wg1_ref, wg2_ref, bg_ref, wa1_ref, wa2_ref, ba_ref,
               out_ref):
    aggr = jnp.maximum(m_ref[...] + tg_ref[...], 0.0)
    cu = jnp.dot(glob_ref[...], wu3_ref[...], preferred_element_type=jnp.float32)
    cu = cu + bu_ref[...][None, :]
    upd = jnp.dot(nodes_ref[...], wu1_ref[...], preferred_element_type=jnp.float32)
    upd = upd + jnp.dot(aggr, wu2_ref[...], preferred_element_type=jnp.float32)
    upd = jnp.maximum(upd + cu, 0.0)
    pooled = jnp.max(upd, axis=0, keepdims=True)
    gemb = jnp.dot(pooled, wg1_ref[...], preferred_element_type=jnp.float32)
    gemb = gemb + jnp.dot(glob_ref[...], wg2_ref[...],
                          preferred_element_type=jnp.float32)
    gemb = jnp.maximum(gemb + bg_ref[...][None, :], 0.0)
    abias = jnp.dot(gemb, wa2_ref[...], preferred_element_type=jnp.float32)
    abias = abias + ba_ref[...][None, :]
    logits = jnp.dot(upd, wa1_ref[...], preferred_element_type=jnp.float32)
    logits = logits + abias
    mx = jnp.max(logits, axis=1, keepdims=True)
    ex = jnp.exp(logits - mx)
    out_ref[...] = ex / jnp.sum(ex, axis=1, keepdims=True)


def kernel(nodes, edge_indices, edge_attr, global_attr, num_nodes, num_edges,
           batch_indices, target_batch_wide,
           W_msg, b_msg, W_upd, b_upd, W_glob, b_glob, W_act, b_act):
    src = edge_indices[0]
    dst = edge_indices[1]
    w13 = jnp.concatenate([W_msg[0:128, :], W_msg[144:272, :]], axis=1)
    w2 = W_msg[128:144, :]
    w4 = W_msg[272:288, :]

    s, tg = pl.pallas_call(
        _prep_body,
        out_shape=(jax.ShapeDtypeStruct((N, DM), jnp.float32),
                   jax.ShapeDtypeStruct((N, DM), jnp.float32)),
    )(nodes, global_attr, w13, w4, b_msg)

    BE = 4000
    a = pl.pallas_call(
        _edgeproj_body,
        grid=(E // BE,),
        in_specs=[pl.BlockSpec((BE, 16), lambda i: (i, 0)),
                  pl.BlockSpec((16, DM), lambda i: (0, 0))],
        out_specs=pl.BlockSpec((BE, DM), lambda i: (i, 0)),
        out_shape=jax.ShapeDtypeStruct((E, DM), jnp.float32),
    )(edge_attr, w2)

    z = jnp.take(s, src, axis=0) + a
    m = jax.ops.segment_max(z, dst, num_segments=N)
    m = jnp.where(jnp.isfinite(m), m, NEG)

    out = pl.pallas_call(
        _post_body,
        out_shape=jax.ShapeDtypeStruct((N, 8), jnp.float32),
    )(nodes, m, tg, global_attr,
      W_upd[0:128, :], W_upd[128:256, :], W_upd[256:272, :], b_upd,
      W_glob[0:128, :], W_glob[128:144, :], b_glob,
      W_act[0:128, :], W_act[128:160, :], b_act)
    return out


# speedup vs baseline: 2.1279x; 1.4658x over previous
import functools

import jax
import jax.numpy as jnp
from jax import lax
from jax.experimental import pallas as pl
from jax.experimental.pallas import tpu as pltpu
from jax.experimental.pallas import tpu_sc as plsc

N = 10000
E = 320000
DF = 128
DM = 128
NEG = -3.0e38

NWORK = 32
RPT = 313
NPAD = NWORK * RPT
CH = 3200
NCH = E // CH
GB = 128
DUMMY = RPT * DM


def _permute(vec, idx):
    dnums = lax.GatherDimensionNumbers(
        offset_dims=(), collapsed_slice_dims=(0,), start_index_map=(0,))
    return lax.gather(vec, idx[:, None], dnums, (1,),
                      mode=lax.GatherScatterMode.PROMISE_IN_BOUNDS)


def _bcast_lane(vec, l):
    return _permute(vec, jnp.full((16,), l, jnp.int32))


def _scmax_body(s_hbm, a_hbm, src_hbm, dst_hbm, m_hbm,
                m_loc, dstc, srcc, sel_e, sel_s, sel_b,
                sbuf, abuf, sems, sema):
    wid = lax.axis_index("c") * 16 + lax.axis_index("s")
    lo = wid * RPT
    iota = lax.iota(jnp.int32, 16)

    def init_m(i, carry):
        m_loc[pl.ds(i * 16, 16)] = jnp.full((16,), NEG, jnp.float32)
        return carry

    lax.fori_loop(0, (RPT * DM + DM) // 16, init_m, 0)

    def init_sel(i, carry):
        z16 = jnp.zeros((16,), jnp.int32)
        sel_e[pl.ds(i * 16, 16)] = z16
        sel_s[pl.ds(i * 16, 16)] = z16
        return carry

    lax.fori_loop(0, CH // 16, init_sel, 0)

    def chunk_body(c, carry):
        pltpu.sync_copy(dst_hbm.at[pl.ds(c * CH, CH)], dstc)
        pltpu.sync_copy(src_hbm.at[pl.ds(c * CH, CH)], srcc)

        def scan_body(v, cnt):
            dv = dstc[pl.ds(v * 16, 16)]
            sv = srcc[pl.ds(v * 16, 16)]
            dloc = dv - lo
            msk = (dloc >= 0) & (dloc < RPT)
            key = jnp.where(msk, iota, iota + 16)
            _, perm = plsc.sort_key_val(key, iota)
            sel_e[pl.ds(cnt, 16)] = perm + (c * CH + v * 16)
            sel_s[pl.ds(cnt, 16)] = _permute(sv, perm)
            sel_b[pl.ds(cnt, 16)] = _permute(dloc * DM, perm)
            return cnt + jnp.sum(msk.astype(jnp.int32))

        k = lax.fori_loop(0, CH // 16, scan_body, jnp.int32(0))
        sel_b[pl.ds(k, 16)] = jnp.full((16,), DUMMY, jnp.int32)
        p = (k + 15) & ~15
        ng = (p + GB - 1) // GB

        def group_body(g, carry2):
            cs = pltpu.async_copy(s_hbm.at[sel_s.at[pl.ds(g * GB, GB)]],
                                  sbuf, sems)
            ca = pltpu.async_copy(a_hbm.at[sel_e.at[pl.ds(g * GB, GB)]],
                                  abuf, sema)
            cs.wait()
            ca.wait()
            nv = jnp.clip((p - g * GB) // 16, 0, GB // 16)

            def vec_body(v, carry3):
                bv16 = sel_b[pl.ds(g * GB + v * 16, 16)]
                for l in range(16):
                    bs = _bcast_lane(bv16, l)
                    jrow = v * 16 + l
                    for cc in range(DM // 16):
                        zv = (sbuf[jrow, pl.ds(cc * 16, 16)]
                              + abuf[jrow, pl.ds(cc * 16, 16)])
                        idx = bs + (cc * 16) + iota
                        cur = plsc.load_gather(m_loc, [idx])
                        plsc.store_scatter(m_loc, [idx],
                                           jnp.maximum(cur, zv))
                return carry3

            lax.fori_loop(0, nv, vec_body, 0)
            return carry2

        lax.fori_loop(0, ng, group_body, 0)
        return carry

    lax.fori_loop(0, NCH, chunk_body, 0)
    pltpu.sync_copy(m_loc.at[pl.ds(0, RPT * DM)],
                    m_hbm.at[pl.ds(lo * DM, RPT * DM)])


def _scatter_max(s, a, src, dst):
    mesh = plsc.VectorSubcoreMesh(core_axis_name="c", subcore_axis_name="s")
    kfn = pl.kernel(
        _scmax_body,
        out_type=jax.ShapeDtypeStruct((NPAD * DM,), jnp.float32),
        mesh=mesh,
        compiler_params=pltpu.CompilerParams(needs_layout_passes=False),
        scratch_types=[
            pltpu.VMEM((RPT * DM + DM,), jnp.float32),
            pltpu.VMEM((CH,), jnp.int32),
            pltpu.VMEM((CH,), jnp.int32),
            pltpu.VMEM((CH,), jnp.int32),
            pltpu.VMEM((CH,), jnp.int32),
            pltpu.VMEM((CH + 16,), jnp.int32),
            pltpu.VMEM((GB, DM), jnp.float32),
            pltpu.VMEM((GB, DM), jnp.float32),
            pltpu.SemaphoreType.DMA,
            pltpu.SemaphoreType.DMA,
        ],
    )
    m_flat = kfn(s, a, src, dst)
    return m_flat.reshape(NPAD, DM)[:N]


def _prep_body(nodes_ref, glob_ref, w13_ref, w4_ref, bmsg_ref, s_ref, tg_ref):
    sm = jnp.dot(nodes_ref[...], w13_ref[...], preferred_element_type=jnp.float32)
    g = jnp.dot(glob_ref[...], w4_ref[...], preferred_element_type=jnp.float32)
    g = g + bmsg_ref[...][None, :]
    s_ref[...] = sm[:, :DM]
    tg_ref[...] = sm[:, DM:] + g


def _edgeproj_body(ea_ref, w2_ref, a_ref):
    a_ref[...] = jnp.dot(ea_ref[...], w2_ref[...], preferred_element_type=jnp.float32)


def _post_body(nodes_ref, m_ref, tg_ref, glob_ref, wu1_ref, wu2_ref, wu3_ref,
               bu_ref, wg1_ref, wg2_ref, bg_ref, wa1_ref, wa2_ref, ba_ref,
               out_ref):
    aggr = jnp.maximum(m_ref[...] + tg_ref[...], 0.0)
    cu = jnp.dot(glob_ref[...], wu3_ref[...], preferred_element_type=jnp.float32)
    cu = cu + bu_ref[...][None, :]
    upd = jnp.dot(nodes_ref[...], wu1_ref[...], preferred_element_type=jnp.float32)
    upd = upd + jnp.dot(aggr, wu2_ref[...], preferred_element_type=jnp.float32)
    upd = jnp.maximum(upd + cu, 0.0)
    pooled = jnp.max(upd, axis=0, keepdims=True)
    gemb = jnp.dot(pooled, wg1_ref[...], preferred_element_type=jnp.float32)
    gemb = gemb + jnp.dot(glob_ref[...], wg2_ref[...],
                          preferred_element_type=jnp.float32)
    gemb = jnp.maximum(gemb + bg_ref[...][None, :], 0.0)
    abias = jnp.dot(gemb, wa2_ref[...], preferred_element_type=jnp.float32)
    abias = abias + ba_ref[...][None, :]
    logits = jnp.dot(upd, wa1_ref[...], preferred_element_type=jnp.float32)
    logits = logits + abias
    mx = jnp.max(logits, axis=1, keepdims=True)
    ex = jnp.exp(logits - mx)
    out_ref[...] = ex / jnp.sum(ex, axis=1, keepdims=True)


def kernel(nodes, edge_indices, edge_attr, global_attr, num_nodes, num_edges,
           batch_indices, target_batch_wide,
           W_msg, b_msg, W_upd, b_upd, W_glob, b_glob, W_act, b_act):
    src = edge_indices[0]
    dst = edge_indices[1]
    w13 = jnp.concatenate([W_msg[0:128, :], W_msg[144:272, :]], axis=1)
    w2 = W_msg[128:144, :]
    w4 = W_msg[272:288, :]

    s, tg = pl.pallas_call(
        _prep_body,
        out_shape=(jax.ShapeDtypeStruct((N, DM), jnp.float32),
                   jax.ShapeDtypeStruct((N, DM), jnp.float32)),
    )(nodes, global_attr, w13, w4, b_msg)

    BE = 4000
    a = pl.pallas_call(
        _edgeproj_body,
        grid=(E // BE,),
        in_specs=[pl.BlockSpec((BE, 16), lambda i: (i, 0)),
                  pl.BlockSpec((16, DM), lambda i: (0, 0))],
        out_specs=pl.BlockSpec((BE, DM), lambda i: (i, 0)),
        out_shape=jax.ShapeDtypeStruct((E, DM), jnp.float32),
    )(edge_attr, w2)

    m = _scatter_max(s, a, src, dst)

    out = pl.pallas_call(
        _post_body,
        out_shape=jax.ShapeDtypeStruct((N, 8), jnp.float32),
    )(nodes, m, tg, global_attr,
      W_upd[0:128, :], W_upd[128:256, :], W_upd[256:272, :], b_upd,
      W_glob[0:128, :], W_glob[128:144, :], b_glob,
      W_act[0:128, :], W_act[128:160, :], b_act)
    return out


# speedup vs baseline: 2.5165x; 1.1827x over previous
import functools

import jax
import jax.numpy as jnp
from jax import lax
from jax.experimental import pallas as pl
from jax.experimental.pallas import tpu as pltpu
from jax.experimental.pallas import tpu_sc as plsc

N = 10000
E = 320000
DF = 128
DM = 128
NEG = -3.0e38

NWORK = 32
RPT = 313
NPAD = NWORK * RPT
CH = 1600
NCH = E // CH
GB = 64
DUMMY = RPT * DM


def _permute(vec, idx):
    dnums = lax.GatherDimensionNumbers(
        offset_dims=(), collapsed_slice_dims=(0,), start_index_map=(0,))
    return lax.gather(vec, idx[:, None], dnums, (1,),
                      mode=lax.GatherScatterMode.PROMISE_IN_BOUNDS)


def _bcast_lane(vec, l):
    return _permute(vec, jnp.full((16,), l, jnp.int32))


def _scmax_body(s_hbm, a_hbm, src_hbm, dst_hbm, m_hbm,
                m_loc, dstc0, srcc0, dstc1, srcc1,
                sel_e0, sel_s0, sel_b0, sel_e1, sel_s1, sel_b1,
                sbuf0, abuf0, sbuf1, abuf1,
                sld0, sld1, sgs0, sga0, sgs1, sga1):
    wid = lax.axis_index("c") * 16 + lax.axis_index("s")
    lo = wid * RPT
    iota = lax.iota(jnp.int32, 16)
    banks = ((dstc0, srcc0, sel_e0, sel_s0, sel_b0, sbuf0, abuf0,
              sld0, sgs0, sga0),
             (dstc1, srcc1, sel_e1, sel_s1, sel_b1, sbuf1, abuf1,
              sld1, sgs1, sga1))

    def init_m(i, carry):
        m_loc[pl.ds(i * 16, 16)] = jnp.full((16,), NEG, jnp.float32)
        return carry

    lax.fori_loop(0, (RPT * DM + DM) // 16, init_m, 0)

    def init_sel(i, carry):
        z16 = jnp.zeros((16,), jnp.int32)
        sel_e0[pl.ds(i * 16, 16)] = z16
        sel_s0[pl.ds(i * 16, 16)] = z16
        sel_e1[pl.ds(i * 16, 16)] = z16
        sel_s1[pl.ds(i * 16, 16)] = z16
        return carry

    lax.fori_loop(0, CH // 16, init_sel, 0)

    def fire_load(c, bank):
        dstc, srcc = bank[0], bank[1]
        pltpu.make_async_copy(dst_hbm.at[pl.ds(c * CH, CH)], dstc,
                              bank[7]).start()
        pltpu.make_async_copy(src_hbm.at[pl.ds(c * CH, CH)], srcc,
                              bank[7]).start()

    def wait_load(c, bank):
        pltpu.make_async_copy(dst_hbm.at[pl.ds(c * CH, CH)], bank[0],
                              bank[7]).wait()
        pltpu.make_async_copy(src_hbm.at[pl.ds(c * CH, CH)], bank[1],
                              bank[7]).wait()

    def fire_gather(g, bank):
        sel_s, sel_e = bank[3], bank[2]
        pltpu.make_async_copy(s_hbm.at[sel_s.at[pl.ds(g * GB, GB)]],
                              bank[5], bank[8]).start()
        pltpu.make_async_copy(a_hbm.at[sel_e.at[pl.ds(g * GB, GB)]],
                              bank[6], bank[9]).start()

    def wait_gather(g, bank):
        sel_s, sel_e = bank[3], bank[2]
        pltpu.make_async_copy(s_hbm.at[sel_s.at[pl.ds(g * GB, GB)]],
                              bank[5], bank[8]).wait()
        pltpu.make_async_copy(a_hbm.at[sel_e.at[pl.ds(g * GB, GB)]],
                              bank[6], bank[9]).wait()

    def scan_chunk(c, bank):
        dstc, srcc, sel_e, sel_s, sel_b = bank[:5]

        def scan_one(v, cnt):
            dv = dstc[pl.ds(v * 16, 16)]
            sv = srcc[pl.ds(v * 16, 16)]
            dloc = dv - lo
            msk = dloc.astype(jnp.uint32) < jnp.uint32(RPT)
            key = jnp.where(msk, iota, iota + 16)
            _, perm = plsc.sort_key_val(key, iota)
            sel_e[pl.ds(cnt, 16)] = perm + (c * CH + v * 16)
            sel_s[pl.ds(cnt, 16)] = _permute(sv, perm)
            sel_b[pl.ds(cnt, 16)] = _permute(dloc * DM, perm)
            return cnt + jnp.sum(msk.astype(jnp.int32))

        def scan4(u, cnt):
            for q in range(4):
                cnt = scan_one(u * 4 + q, cnt)
            return cnt

        k = lax.fori_loop(0, CH // 64, scan4, jnp.int32(0))
        sel_b[pl.ds(k, 16)] = jnp.full((16,), DUMMY, jnp.int32)
        return k

    def update_chunk(k, bank):
        sel_b, sbuf, abuf = bank[4], bank[5], bank[6]
        p = (k + 15) & ~15
        ng = (p + GB - 1) // GB

        def group_body(g, carry2):
            @pl.when(g > 0)
            def _():
                fire_gather(g, bank)

            wait_gather(g, bank)
            nv = jnp.clip((p - g * GB) // 16, 0, GB // 16)

            def vec_body(v, carry3):
                bv16 = sel_b[pl.ds(g * GB + v * 16, 16)]
                for l in range(16):
                    bs = _bcast_lane(bv16, l)
                    jrow = v * 16 + l
                    for cc in range(DM // 16):
                        zv = (sbuf[jrow, pl.ds(cc * 16, 16)]
                              + abuf[jrow, pl.ds(cc * 16, 16)])
                        idx = bs + (cc * 16) + iota
                        cur = plsc.load_gather(m_loc, [idx])
                        plsc.store_scatter(m_loc, [idx],
                                           jnp.maximum(cur, zv))
                return carry3

            lax.fori_loop(0, nv, vec_body, 0)
            return carry2

        lax.fori_loop(0, ng, group_body, 0)

    def do_chunk(c, bank, other, k_prev):
        wait_load(c, bank)
        k = scan_chunk(c, bank)
        fire_gather(0, bank)

        @pl.when(c + 1 < NCH)
        def _():
            fire_load(c + 1, other)

        @pl.when(c > 0)
        def _():
            update_chunk(k_prev, other)

        return k

    fire_load(0, banks[0])

    def pair_body(i, k_prev):
        k0 = do_chunk(2 * i, banks[0], banks[1], k_prev)
        k1 = do_chunk(2 * i + 1, banks[1], banks[0], k0)
        return k1

    k_last = lax.fori_loop(0, NCH // 2, pair_body, jnp.int32(0))
    update_chunk(k_last, banks[1])
    pltpu.sync_copy(m_loc.at[pl.ds(0, RPT * DM)],
                    m_hbm.at[pl.ds(lo * DM, RPT * DM)])


def _scatter_max(s, a, src, dst):
    mesh = plsc.VectorSubcoreMesh(core_axis_name="c", subcore_axis_name="s")
    kfn = pl.kernel(
        _scmax_body,
        out_type=jax.ShapeDtypeStruct((NPAD * DM,), jnp.float32),
        mesh=mesh,
        compiler_params=pltpu.CompilerParams(needs_layout_passes=False),
        scratch_types=[
            pltpu.VMEM((RPT * DM + DM,), jnp.float32),
            pltpu.VMEM((CH,), jnp.int32),
            pltpu.VMEM((CH,), jnp.int32),
            pltpu.VMEM((CH,), jnp.int32),
            pltpu.VMEM((CH,), jnp.int32),
            pltpu.VMEM((CH,), jnp.int32),
            pltpu.VMEM((CH,), jnp.int32),
            pltpu.VMEM((CH + 16,), jnp.int32),
            pltpu.VMEM((CH,), jnp.int32),
            pltpu.VMEM((CH,), jnp.int32),
            pltpu.VMEM((CH + 16,), jnp.int32),
            pltpu.VMEM((GB, DM), jnp.float32),
            pltpu.VMEM((GB, DM), jnp.float32),
            pltpu.VMEM((GB, DM), jnp.float32),
            pltpu.VMEM((GB, DM), jnp.float32),
            pltpu.SemaphoreType.DMA,
            pltpu.SemaphoreType.DMA,
            pltpu.SemaphoreType.DMA,
            pltpu.SemaphoreType.DMA,
            pltpu.SemaphoreType.DMA,
            pltpu.SemaphoreType.DMA,
        ],
    )
    m_flat = kfn(s, a, src, dst)
    return m_flat.reshape(NPAD, DM)[:N]


def _prep_body(nodes_ref, glob_ref, w13_ref, w4_ref, bmsg_ref, s_ref, tg_ref):
    sm = jnp.dot(nodes_ref[...], w13_ref[...], preferred_element_type=jnp.float32)
    g = jnp.dot(glob_ref[...], w4_ref[...], preferred_element_type=jnp.float32)
    g = g + bmsg_ref[...][None, :]
    s_ref[...] = sm[:, :DM]
    tg_ref[...] = sm[:, DM:] + g


def _edgeproj_body(ea_ref, w2_ref, a_ref):
    a_ref[...] = jnp.dot(ea_ref[...], w2_ref[...], preferred_element_type=jnp.float32)


def _post_body(nodes_ref, m_ref, tg_ref, glob_ref, wu1_ref, wu2_ref, wu3_ref,
               bu_ref, wg1_ref, wg2_ref, bg_ref, wa1_ref, wa2_ref, ba_ref,
               out_ref):
    aggr = jnp.maximum(m_ref[...] + tg_ref[...], 0.0)
    cu = jnp.dot(glob_ref[...], wu3_ref[...], preferred_element_type=jnp.float32)
    cu = cu + bu_ref[...][None, :]
    upd = jnp.dot(nodes_ref[...], wu1_ref[...], preferred_element_type=jnp.float32)
    upd = upd + jnp.dot(aggr, wu2_ref[...], preferred_element_type=jnp.float32)
    upd = jnp.maximum(upd + cu, 0.0)
    pooled = jnp.max(upd, axis=0, keepdims=True)
    gemb = jnp.dot(pooled, wg1_ref[...], preferred_element_type=jnp.float32)
    gemb = gemb + jnp.dot(glob_ref[...], wg2_ref[...],
                          preferred_element_type=jnp.float32)
    gemb = jnp.maximum(gemb + bg_ref[...][None, :], 0.0)
    abias = jnp.dot(gemb, wa2_ref[...], preferred_element_type=jnp.float32)
    abias = abias + ba_ref[...][None, :]
    logits = jnp.dot(upd, wa1_ref[...], preferred_element_type=jnp.float32)
    logits = logits + abias
    mx = jnp.max(logits, axis=1, keepdims=True)
    ex = jnp.exp(logits - mx)
    out_ref[...] = ex / jnp.sum(ex, axis=1, keepdims=True)


def kernel(nodes, edge_indices, edge_attr, global_attr, num_nodes, num_edges,
           batch_indices, target_batch_wide,
           W_msg, b_msg, W_upd, b_upd, W_glob, b_glob, W_act, b_act):
    src = edge_indices[0]
    dst = edge_indices[1]
    w13 = jnp.concatenate([W_msg[0:128, :], W_msg[144:272, :]], axis=1)
    w2 = W_msg[128:144, :]
    w4 = W_msg[272:288, :]

    s, tg = pl.pallas_call(
        _prep_body,
        out_shape=(jax.ShapeDtypeStruct((N, DM), jnp.float32),
                   jax.ShapeDtypeStruct((N, DM), jnp.float32)),
    )(nodes, global_attr, w13, w4, b_msg)

    BE = 4000
    a = pl.pallas_call(
        _edgeproj_body,
        grid=(E // BE,),
        in_specs=[pl.BlockSpec((BE, 16), lambda i: (i, 0)),
                  pl.BlockSpec((16, DM), lambda i: (0, 0))],
        out_specs=pl.BlockSpec((BE, DM), lambda i: (i, 0)),
        out_shape=jax.ShapeDtypeStruct((E, DM), jnp.float32),
    )(edge_attr, w2)

    m = _scatter_max(s, a, src, dst)

    out = pl.pallas_call(
        _post_body,
        out_shape=jax.ShapeDtypeStruct((N, 8), jnp.float32),
    )(nodes, m, tg, global_attr,
      W_upd[0:128, :], W_upd[128:256, :], W_upd[256:272, :], b_upd,
      W_glob[0:128, :], W_glob[128:144, :], b_glob,
      W_act[0:128, :], W_act[128:160, :], b_act)
    return out
